# Initial kernel scaffold; baseline (speedup 1.0000x reference)
import jax
import jax.numpy as jnp
from jax.experimental import pallas as pl


def kernel(img, mask, l1_wd, l1_wi, n_w, l2_wd, l2_wi):
    raise NotImplementedError("write your pallas kernel here")


# speedup vs baseline: 1.4279x; 1.4279x over previous
import jax
import jax.numpy as jnp
from jax.experimental import pallas as pl
from jax.experimental.pallas import tpu as pltpu


def _softplus(x):
    return jnp.maximum(x, 0.0) + jnp.log(1.0 + jnp.exp(-jnp.abs(x)))


def _gcnet_kernel(
        img_ref,
        mask_ref,
        w1_ref,
        b1_ref,
        g3_ref,
        eye_ref,
        wn_ref,
        wnd_ref,
        p_ref,
        q_ref,
        w2_ref,
        b2m_ref,
        w2n_ref,
        w2s_ref,
        dirs1_ref,
        intens1_ref,
        normal_ref,
        shading_ref,
        dirs2_ref,
        intens2_ref):
    C = img_ref.shape[0]
    img = img_ref[...]
    mask = mask_ref[...]

    pool = jnp.mean(img, axis=-1, keepdims=True)
    mpool = jnp.mean(mask, axis=-1, keepdims=True)

    hi = jax.lax.Precision.HIGHEST

    h1 = (jnp.dot(w1_ref[...], pool, preferred_element_type=jnp.float32)
          + b1_ref[...] * mpool)
    d1, i1 = h1[0:C], h1[C:2 * C]
    nrm1 = jnp.sqrt(jnp.dot(g3_ref[...], d1 * d1, precision=hi,
                            preferred_element_type=jnp.float32))
    dirs1 = d1 / (nrm1 + 1e-8)
    intens1 = _softplus(i1) + 0.2
    dirs1_ref[...] = dirs1
    intens1_ref[...] = intens1

    inv = pl.reciprocal(intens1 + 1e-8, approx=True)
    dir_col = jnp.dot(wnd_ref[...], dirs1,
                      preferred_element_type=jnp.float32)
    raw = jnp.dot(wn_ref[...], img * inv,
                  preferred_element_type=jnp.float32) + dir_col
    t = jnp.tanh(raw)
    scale = jax.lax.rsqrt(jnp.sum(t * t, axis=0, keepdims=True) + 1e-8) * mask
    normal = t * scale
    normal_ref[...] = normal[0:3, :]

    dmat = jnp.dot(
        jnp.dot(p_ref[...], eye_ref[...] * dirs1, precision=hi,
                preferred_element_type=jnp.float32),
        q_ref[...], precision=hi,
        preferred_element_type=jnp.float32)
    shad = (dmat[:, 0:1] * normal[0:1, :]
            + dmat[:, 1:2] * normal[1:2, :]
            + dmat[:, 2:3] * normal[2:3, :])
    shad = jnp.clip(shad, 0.0, 1.0)
    shading_ref[...] = shad

    sp = jnp.mean(shad, axis=-1, keepdims=True)
    npool = jnp.mean(normal, axis=-1, keepdims=True)
    h2 = (jnp.dot(w2_ref[...], pool, preferred_element_type=jnp.float32)
          + b2m_ref[...] * mpool
          + jnp.dot(w2n_ref[...], npool, preferred_element_type=jnp.float32)
          + jnp.dot(w2s_ref[...], sp, precision=hi,
                    preferred_element_type=jnp.float32))
    d2, i2 = h2[0:C], h2[C:2 * C]
    nrm2 = jnp.sqrt(jnp.dot(g3_ref[...], d2 * d2, precision=hi,
                            preferred_element_type=jnp.float32))
    dirs2_ref[...] = d2 / (nrm2 + 1e-8)
    intens2_ref[...] = _softplus(i2) + 0.2


def kernel(img, mask, l1_wd, l1_wi, n_w, l2_wd, l2_wi):
    N, c3f, H, W = img.shape
    F = c3f // 3
    C = 3 * F
    HW = H * W
    f32 = jnp.float32
    img_r = img.reshape(N, C, HW).astype(f32)
    mask_r = mask.reshape(N, 1, HW).astype(f32)

    eyeF = jnp.eye(F, dtype=f32)
    eye3 = jnp.eye(3, dtype=f32)
    w1 = jnp.concatenate([jnp.kron(eyeF, l1_wd[:, 0:3]),
                          jnp.kron(eyeF, l1_wi[:, 0:3])], axis=0)
    b1 = jnp.concatenate([jnp.tile(l1_wd[:, 3:4], (F, 1)),
                          jnp.tile(l1_wi[:, 3:4], (F, 1))], axis=0)
    g3 = jnp.kron(eyeF, jnp.ones((3, 3), f32))
    eye24 = jnp.eye(C, dtype=f32)

    n_wr = n_w.reshape(3, F, 6)
    zpad = jnp.zeros((5, C), f32)
    wn = jnp.concatenate([n_wr[:, :, 0:3].reshape(3, C), zpad], axis=0)
    wnd = jnp.concatenate([n_wr[:, :, 3:6].reshape(3, C), zpad], axis=0)

    pmat = jnp.kron(eyeF, jnp.ones((1, 3), f32))
    qmat = jnp.concatenate([jnp.kron(jnp.ones((F, 1), f32), eye3),
                            jnp.zeros((C, 5), f32)], axis=1)

    w2 = jnp.concatenate([jnp.kron(eyeF, l2_wd[:, 0:3]),
                          jnp.kron(eyeF, l2_wi[:, 0:3])], axis=0)
    b2m = jnp.concatenate([jnp.tile(l2_wd[:, 3:4], (F, 1)),
                           jnp.tile(l2_wi[:, 3:4], (F, 1))], axis=0)
    zc5 = jnp.zeros((C, 5), f32)
    w2n = jnp.concatenate([
        jnp.concatenate([jnp.tile(l2_wd[:, 4:7], (F, 1)), zc5], axis=1),
        jnp.concatenate([jnp.tile(l2_wi[:, 4:7], (F, 1)), zc5], axis=1)],
        axis=0)
    w2s = jnp.concatenate([jnp.kron(eyeF, l2_wd[:, 7:8]),
                           jnp.kron(eyeF, l2_wi[:, 7:8])], axis=0)

    def cspec(shape):
        return pl.BlockSpec(shape, lambda n: (0,) * len(shape))

    outs = pl.pallas_call(
        _gcnet_kernel,
        grid=(N,),
        in_specs=[
            pl.BlockSpec((None, C, HW), lambda n: (n, 0, 0)),
            pl.BlockSpec((None, 1, HW), lambda n: (n, 0, 0)),
            cspec((2 * C, C)), cspec((2 * C, 1)),
            cspec((C, C)), cspec((C, C)),
            cspec((8, C)), cspec((8, C)),
            cspec((F, C)), cspec((C, 8)),
            cspec((2 * C, C)), cspec((2 * C, 1)),
            cspec((2 * C, 8)), cspec((2 * C, 8)),
        ],
        out_specs=[
            pl.BlockSpec((None, C, 1), lambda n: (n, 0, 0)),
            pl.BlockSpec((None, C, 1), lambda n: (n, 0, 0)),
            pl.BlockSpec((None, 3, HW), lambda n: (n, 0, 0)),
            pl.BlockSpec((None, F, HW), lambda n: (n, 0, 0)),
            pl.BlockSpec((None, C, 1), lambda n: (n, 0, 0)),
            pl.BlockSpec((None, C, 1), lambda n: (n, 0, 0)),
        ],
        out_shape=[
            jax.ShapeDtypeStruct((N, C, 1), f32),
            jax.ShapeDtypeStruct((N, C, 1), f32),
            jax.ShapeDtypeStruct((N, 3, HW), f32),
            jax.ShapeDtypeStruct((N, F, HW), f32),
            jax.ShapeDtypeStruct((N, C, 1), f32),
            jax.ShapeDtypeStruct((N, C, 1), f32),
        ],
        compiler_params=pltpu.CompilerParams(
            dimension_semantics=("parallel",)),
    )(img_r, mask_r, w1, b1, g3, eye24, wn, wnd, pmat, qmat, w2, b2m, w2n, w2s)

    dirs1, intens1, normal, shading, dirs2, intens2 = outs
    return {
        'prev_dirs': dirs1.reshape(N, F, 3),
        'prev_intens': intens1.reshape(N, F, 3),
        'prev_normal': normal.reshape(N, 3, H, W),
        'prev_shading': shading.reshape(N, F, H, W),
        'dirs': dirs2.reshape(N, F, 3),
        'intens': intens2.reshape(N, F, 3),
    }


# speedup vs baseline: 2.6288x; 1.8410x over previous
import jax
import jax.numpy as jnp
from jax.experimental import pallas as pl
from jax.experimental.pallas import tpu as pltpu

_HI = jax.lax.Precision.HIGHEST


def _softplus(x):
    return jnp.maximum(x, 0.0) + jnp.log(1.0 + jnp.exp(-jnp.abs(x)))


def _gcnet_kernel(
        img_ref,
        mask_ref,
        s24_ref,
        w1_ref,
        b1_ref,
        g3_ref,
        eye_ref,
        kw_ref,
        wnd_ref,
        r24_ref,
        p_ref,
        q_ref,
        w2_ref,
        b2m_ref,
        w2n_ref,
        w2s_ref,
        normal_ref,
        shading_ref,
        small_ref):
    C, H, W = img_ref.shape
    F = C // 3
    inv_hw = 1.0 / (H * W)
    img = img_ref[...]
    mask = mask_ref[...].reshape(H, W)

    pw = jnp.dot(s24_ref[...], img.reshape(C * H, W),
                 preferred_element_type=jnp.float32)
    pool = jnp.sum(pw, axis=-1, keepdims=True) * inv_hw
    mpool = jnp.mean(mask, keepdims=True)

    h1 = (jnp.dot(w1_ref[...], pool, preferred_element_type=jnp.float32)
          + b1_ref[...] * mpool)
    d1, i1 = h1[0:C], h1[C:2 * C]
    nrm1 = jnp.sqrt(jnp.dot(g3_ref[...], d1 * d1, precision=_HI,
                            preferred_element_type=jnp.float32))
    dirs1 = d1 / (nrm1 + 1e-8)
    intens1 = _softplus(i1) + 0.2

    inv = pl.reciprocal(intens1 + 1e-8, approx=True)
    imgn = img * inv.reshape(C, 1, 1)
    dir3 = jnp.dot(wnd_ref[...], dirs1,
                   preferred_element_type=jnp.float32)
    dir24 = jnp.dot(r24_ref[...], dir3, precision=_HI,
                    preferred_element_type=jnp.float32)

    dmat = jnp.dot(jnp.dot(p_ref[...], eye_ref[...] * dirs1, precision=_HI,
                           preferred_element_type=jnp.float32),
                   q_ref[...], precision=_HI,
                   preferred_element_type=jnp.float32)
    dm0 = dmat[:, 0:1].reshape(F, 1, 1)
    dm1 = dmat[:, 1:2].reshape(F, 1, 1)
    dm2 = dmat[:, 2:3].reshape(F, 1, 1)

    np_acc = jnp.zeros((3, 1, 1), jnp.float32)
    sp_acc = jnp.zeros((F, 1, 1), jnp.float32)
    for k in range(H // 8):
        sl = slice(k * 8, (k + 1) * 8)
        chunk = imgn[:, sl, :].reshape(C * 8, W)
        raw = (jnp.dot(kw_ref[...], chunk,
                       preferred_element_type=jnp.float32)
               + dir24)
        t3 = jnp.tanh(raw).reshape(3, 8, W)
        ssum = jnp.sum(t3 * t3, axis=0)
        scale = jax.lax.rsqrt(ssum + 1e-8) * mask[sl, :]
        normal_k = t3 * scale
        normal_ref[:, sl, :] = normal_k
        np_acc = np_acc + jnp.sum(normal_k, axis=(1, 2), keepdims=True)

        shad_k = (dm0 * normal_k[0:1] + dm1 * normal_k[1:2]
                  + dm2 * normal_k[2:3])
        shad_k = jnp.clip(shad_k, 0.0, 1.0)
        shading_ref[:, sl, :] = shad_k
        sp_acc = sp_acc + jnp.sum(shad_k, axis=(1, 2), keepdims=True)

    npool = np_acc.reshape(3, 1) * inv_hw
    sp = sp_acc.reshape(F, 1) * inv_hw

    h2 = (jnp.dot(w2_ref[...], pool, preferred_element_type=jnp.float32)
          + b2m_ref[...] * mpool
          + jnp.dot(w2n_ref[...], npool, preferred_element_type=jnp.float32)
          + jnp.dot(w2s_ref[...], sp, precision=_HI,
                    preferred_element_type=jnp.float32))
    d2, i2 = h2[0:C], h2[C:2 * C]
    nrm2 = jnp.sqrt(jnp.dot(g3_ref[...], d2 * d2, precision=_HI,
                            preferred_element_type=jnp.float32))
    dirs2 = d2 / (nrm2 + 1e-8)
    intens2 = _softplus(i2) + 0.2

    def to_mat(v):
        return jnp.dot(jnp.dot(p_ref[...], eye_ref[...] * v, precision=_HI,
                               preferred_element_type=jnp.float32),
                       q_ref[...], precision=_HI,
                       preferred_element_type=jnp.float32)

    small_ref[...] = jnp.concatenate(
        [dmat, to_mat(intens1), to_mat(dirs2), to_mat(intens2)], axis=1)


def kernel(img, mask, l1_wd, l1_wi, n_w, l2_wd, l2_wi):
    N, c3f, H, W = img.shape
    F = c3f // 3
    C = 3 * F
    f32 = jnp.float32
    img = img.astype(f32)
    mask = mask.astype(f32)

    eyeF = jnp.eye(F, dtype=f32)
    eye3 = jnp.eye(3, dtype=f32)
    s24 = jnp.kron(jnp.eye(C, dtype=f32), jnp.ones((1, H), f32))
    w1 = jnp.concatenate([jnp.kron(eyeF, l1_wd[:, 0:3]),
                          jnp.kron(eyeF, l1_wi[:, 0:3])], axis=0)
    b1 = jnp.concatenate([jnp.tile(l1_wd[:, 3:4], (F, 1)),
                          jnp.tile(l1_wi[:, 3:4], (F, 1))], axis=0)
    g3 = jnp.kron(eyeF, jnp.ones((3, 3), f32))
    eye24 = jnp.eye(C, dtype=f32)

    n_wr = n_w.reshape(3, F, 6)
    wn3 = n_wr[:, :, 0:3].reshape(3, C)
    wnd3 = n_wr[:, :, 3:6].reshape(3, C)
    kw = jnp.kron(wn3, jnp.eye(8, dtype=f32))
    r24 = jnp.kron(eye3, jnp.ones((8, 1), f32))

    pmat = jnp.kron(eyeF, jnp.ones((1, 3), f32))
    qmat = jnp.kron(jnp.ones((F, 1), f32), eye3)

    w2 = jnp.concatenate([jnp.kron(eyeF, l2_wd[:, 0:3]),
                          jnp.kron(eyeF, l2_wi[:, 0:3])], axis=0)
    b2m = jnp.concatenate([jnp.tile(l2_wd[:, 3:4], (F, 1)),
                           jnp.tile(l2_wi[:, 3:4], (F, 1))], axis=0)
    w2n = jnp.concatenate([jnp.tile(l2_wd[:, 4:7], (F, 1)),
                           jnp.tile(l2_wi[:, 4:7], (F, 1))], axis=0)
    w2s = jnp.concatenate([jnp.kron(eyeF, l2_wd[:, 7:8]),
                           jnp.kron(eyeF, l2_wi[:, 7:8])], axis=0)

    def cspec(shape):
        return pl.BlockSpec(shape, lambda n: (0,) * len(shape))

    normal, shading, small = pl.pallas_call(
        _gcnet_kernel,
        grid=(N,),
        in_specs=[
            pl.BlockSpec((None, C, H, W), lambda n: (n, 0, 0, 0)),
            pl.BlockSpec((None, 1, H, W), lambda n: (n, 0, 0, 0)),
            cspec((C, C * H)),
            cspec((2 * C, C)), cspec((2 * C, 1)),
            cspec((C, C)), cspec((C, C)),
            cspec((24, 8 * C)), cspec((3, C)), cspec((C, 3)),
            cspec((F, C)), cspec((C, 3)),
            cspec((2 * C, C)), cspec((2 * C, 1)),
            cspec((2 * C, 3)), cspec((2 * C, F)),
        ],
        out_specs=[
            pl.BlockSpec((None, 3, H, W), lambda n: (n, 0, 0, 0)),
            pl.BlockSpec((None, F, H, W), lambda n: (n, 0, 0, 0)),
            pl.BlockSpec((None, F, 12), lambda n: (n, 0, 0)),
        ],
        out_shape=[
            jax.ShapeDtypeStruct((N, 3, H, W), f32),
            jax.ShapeDtypeStruct((N, F, H, W), f32),
            jax.ShapeDtypeStruct((N, F, 12), f32),
        ],
        compiler_params=pltpu.CompilerParams(
            dimension_semantics=("parallel",)),
    )(img, mask, s24, w1, b1, g3, eye24, kw, wnd3, r24, pmat, qmat,
      w2, b2m, w2n, w2s)

    return {
        'prev_dirs': small[:, :, 0:3],
        'prev_intens': small[:, :, 3:6],
        'prev_normal': normal,
        'prev_shading': shading,
        'dirs': small[:, :, 6:9],
        'intens': small[:, :, 9:12],
    }


# speedup vs baseline: 3.5137x; 1.3366x over previous
import jax
import jax.numpy as jnp
from jax.experimental import pallas as pl
from jax.experimental.pallas import tpu as pltpu

_HI = jax.lax.Precision.HIGHEST


def _softplus(x):
    return jnp.maximum(x, 0.0) + jnp.log(1.0 + jnp.exp(-jnp.abs(x)))


def _gcnet_kernel(
        img_ref,
        mask_ref,
        s24_ref,
        w1_ref,
        b1_ref,
        g3_ref,
        eye_ref,
        kw_ref,
        wnd_ref,
        r24_ref,
        p_ref,
        q_ref,
        w2_ref,
        b2m_ref,
        w2n_ref,
        w2s_ref,
        normal_ref,
        shading_ref,
        small_ref):
    B, C, H, W = img_ref.shape
    F = C // 3
    inv_hw = 1.0 / (H * W)
    img = img_ref[...]
    mask = mask_ref[...].reshape(B * H, W)

    def to_mat(v):
        return jnp.dot(jnp.dot(p_ref[...], eye_ref[...] * v, precision=_HI,
                               preferred_element_type=jnp.float32),
                       q_ref[...], precision=_HI,
                       preferred_element_type=jnp.float32)

    pools, mpools = [], []
    for b in range(B):
        pw = jnp.dot(s24_ref[...], img[b].reshape(C * H, W),
                     preferred_element_type=jnp.float32)
        pools.append(jnp.sum(pw, axis=-1, keepdims=True) * inv_hw)
        mpools.append(jnp.mean(mask[b * H:(b + 1) * H, :], keepdims=True))
    pool = jnp.concatenate(pools, axis=1)
    mpool = jnp.concatenate(mpools, axis=1)

    h1 = (jnp.dot(w1_ref[...], pool, preferred_element_type=jnp.float32)
          + b1_ref[...] * mpool)
    d1, i1 = h1[0:C], h1[C:2 * C]
    nrm1 = jnp.sqrt(jnp.dot(g3_ref[...], d1 * d1, precision=_HI,
                            preferred_element_type=jnp.float32))
    dirs1 = d1 / (nrm1 + 1e-8)
    intens1 = _softplus(i1) + 0.2

    inv = pl.reciprocal(intens1 + 1e-8, approx=True)
    dir3 = jnp.dot(wnd_ref[...], dirs1,
                   preferred_element_type=jnp.float32)
    dir24 = jnp.dot(r24_ref[...], dir3, precision=_HI,
                    preferred_element_type=jnp.float32)

    np_cols, sp_cols, dmats = [], [], []
    for b in range(B):
        imgn = img[b] * inv[:, b:b + 1].reshape(C, 1, 1)
        dcol = dir24[:, b:b + 1]
        dmat = to_mat(dirs1[:, b:b + 1])
        dmats.append(dmat)
        dm0 = dmat[:, 0:1].reshape(F, 1, 1)
        dm1 = dmat[:, 1:2].reshape(F, 1, 1)
        dm2 = dmat[:, 2:3].reshape(F, 1, 1)
        np_acc = jnp.zeros((3, 1, 1), jnp.float32)
        sp_acc = jnp.zeros((F, 1, 1), jnp.float32)
        for k in range(H // 8):
            sl = slice(k * 8, (k + 1) * 8)
            chunk = imgn[:, sl, :].reshape(C * 8, W)
            raw = (jnp.dot(kw_ref[...], chunk,
                           preferred_element_type=jnp.float32)
                   + dcol)
            t3 = jnp.tanh(raw).reshape(3, 8, W)
            ssum = jnp.sum(t3 * t3, axis=0)
            scale = (jax.lax.rsqrt(ssum + 1e-8)
                     * mask[b * H + k * 8:b * H + (k + 1) * 8, :])
            normal_k = t3 * scale
            normal_ref[b, :, sl, :] = normal_k
            np_acc = np_acc + jnp.sum(normal_k, axis=(1, 2), keepdims=True)

            shad_k = (dm0 * normal_k[0:1] + dm1 * normal_k[1:2]
                      + dm2 * normal_k[2:3])
            shad_k = jnp.clip(shad_k, 0.0, 1.0)
            shading_ref[b, :, sl, :] = shad_k
            sp_acc = sp_acc + jnp.sum(shad_k, axis=(1, 2), keepdims=True)
        np_cols.append(np_acc.reshape(3, 1) * inv_hw)
        sp_cols.append(sp_acc.reshape(F, 1) * inv_hw)

    npool = jnp.concatenate(np_cols, axis=1)
    sp = jnp.concatenate(sp_cols, axis=1)

    h2 = (jnp.dot(w2_ref[...], pool, preferred_element_type=jnp.float32)
          + b2m_ref[...] * mpool
          + jnp.dot(w2n_ref[...], npool, preferred_element_type=jnp.float32)
          + jnp.dot(w2s_ref[...], sp, precision=_HI,
                    preferred_element_type=jnp.float32))
    d2, i2 = h2[0:C], h2[C:2 * C]
    nrm2 = jnp.sqrt(jnp.dot(g3_ref[...], d2 * d2, precision=_HI,
                            preferred_element_type=jnp.float32))
    dirs2 = d2 / (nrm2 + 1e-8)
    intens2 = _softplus(i2) + 0.2

    for b in range(B):
        small_ref[b] = jnp.concatenate(
            [dmats[b], to_mat(intens1[:, b:b + 1]),
             to_mat(dirs2[:, b:b + 1]), to_mat(intens2[:, b:b + 1])], axis=1)


def kernel(img, mask, l1_wd, l1_wi, n_w, l2_wd, l2_wi):
    N, c3f, H, W = img.shape
    F = c3f // 3
    C = 3 * F
    f32 = jnp.float32
    img = img.astype(f32)
    mask = mask.astype(f32)

    eyeF = jnp.eye(F, dtype=f32)
    eye3 = jnp.eye(3, dtype=f32)
    s24 = jnp.kron(jnp.eye(C, dtype=f32), jnp.ones((1, H), f32))
    w1 = jnp.concatenate([jnp.kron(eyeF, l1_wd[:, 0:3]),
                          jnp.kron(eyeF, l1_wi[:, 0:3])], axis=0)
    b1 = jnp.concatenate([jnp.tile(l1_wd[:, 3:4], (F, 1)),
                          jnp.tile(l1_wi[:, 3:4], (F, 1))], axis=0)
    g3 = jnp.kron(eyeF, jnp.ones((3, 3), f32))
    eye24 = jnp.eye(C, dtype=f32)

    n_wr = n_w.reshape(3, F, 6)
    wn3 = n_wr[:, :, 0:3].reshape(3, C)
    wnd3 = n_wr[:, :, 3:6].reshape(3, C)
    kw = jnp.kron(wn3, jnp.eye(8, dtype=f32))
    r24 = jnp.kron(eye3, jnp.ones((8, 1), f32))

    pmat = jnp.kron(eyeF, jnp.ones((1, 3), f32))
    qmat = jnp.kron(jnp.ones((F, 1), f32), eye3)

    w2 = jnp.concatenate([jnp.kron(eyeF, l2_wd[:, 0:3]),
                          jnp.kron(eyeF, l2_wi[:, 0:3])], axis=0)
    b2m = jnp.concatenate([jnp.tile(l2_wd[:, 3:4], (F, 1)),
                           jnp.tile(l2_wi[:, 3:4], (F, 1))], axis=0)
    w2n = jnp.concatenate([jnp.tile(l2_wd[:, 4:7], (F, 1)),
                           jnp.tile(l2_wi[:, 4:7], (F, 1))], axis=0)
    w2s = jnp.concatenate([jnp.kron(eyeF, l2_wd[:, 7:8]),
                           jnp.kron(eyeF, l2_wi[:, 7:8])], axis=0)

    def cspec(shape):
        return pl.BlockSpec(shape, lambda n: (0,) * len(shape))

    B = 2 if N % 2 == 0 else 1

    normal, shading, small = pl.pallas_call(
        _gcnet_kernel,
        grid=(N // B,),
        in_specs=[
            pl.BlockSpec((B, C, H, W), lambda n: (n, 0, 0, 0)),
            pl.BlockSpec((B, 1, H, W), lambda n: (n, 0, 0, 0)),
            cspec((C, C * H)),
            cspec((2 * C, C)), cspec((2 * C, 1)),
            cspec((C, C)), cspec((C, C)),
            cspec((24, 8 * C)), cspec((3, C)), cspec((C, 3)),
            cspec((F, C)), cspec((C, 3)),
            cspec((2 * C, C)), cspec((2 * C, 1)),
            cspec((2 * C, 3)), cspec((2 * C, F)),
        ],
        out_specs=[
            pl.BlockSpec((B, 3, H, W), lambda n: (n, 0, 0, 0)),
            pl.BlockSpec((B, F, H, W), lambda n: (n, 0, 0, 0)),
            pl.BlockSpec((B, F, 12), lambda n: (n, 0, 0)),
        ],
        out_shape=[
            jax.ShapeDtypeStruct((N, 3, H, W), f32),
            jax.ShapeDtypeStruct((N, F, H, W), f32),
            jax.ShapeDtypeStruct((N, F, 12), f32),
        ],
        compiler_params=pltpu.CompilerParams(
            dimension_semantics=("parallel",)),
    )(img, mask, s24, w1, b1, g3, eye24, kw, wnd3, r24, pmat, qmat,
      w2, b2m, w2n, w2s)

    return {
        'prev_dirs': small[:, :, 0:3],
        'prev_intens': small[:, :, 3:6],
        'prev_normal': normal,
        'prev_shading': shading,
        'dirs': small[:, :, 6:9],
        'intens': small[:, :, 9:12],
    }


# speedup vs baseline: 4.1259x; 1.1742x over previous
import jax
import jax.numpy as jnp
from jax.experimental import pallas as pl
from jax.experimental.pallas import tpu as pltpu

_HI = jax.lax.Precision.HIGHEST


def _softplus(x):
    return jnp.maximum(x, 0.0) + jnp.log(1.0 + jnp.exp(-jnp.abs(x)))


def _gcnet_kernel(
        img_ref,
        mask_ref,
        s24_ref,
        w1_ref,
        b1_ref,
        g3_ref,
        eye_ref,
        kw_ref,
        wnd_ref,
        r24_ref,
        p_ref,
        q_ref,
        w2_ref,
        b2m_ref,
        w2n_ref,
        w2s_ref,
        normal_ref,
        shading_ref,
        small_ref):
    B, C, H, W = img_ref.shape
    F = C // 3
    inv_hw = 1.0 / (H * W)
    img = img_ref[...]
    mask = mask_ref[...].reshape(B * H, W)

    def to_mat(v):
        return jnp.dot(jnp.dot(p_ref[...], eye_ref[...] * v, precision=_HI,
                               preferred_element_type=jnp.float32),
                       q_ref[...], precision=_HI,
                       preferred_element_type=jnp.float32)

    pools, mpools = [], []
    for b in range(B):
        pw = jnp.dot(s24_ref[...], img[b].reshape(C * H, W),
                     preferred_element_type=jnp.float32)
        pools.append(jnp.sum(pw, axis=-1, keepdims=True) * inv_hw)
        mpools.append(jnp.mean(mask[b * H:(b + 1) * H, :], keepdims=True))
    pool = jnp.concatenate(pools, axis=1)
    mpool = jnp.concatenate(mpools, axis=1)

    h1 = (jnp.dot(w1_ref[...], pool, preferred_element_type=jnp.float32)
          + b1_ref[...] * mpool)
    d1, i1 = h1[0:C], h1[C:2 * C]
    nrm1 = jnp.sqrt(jnp.dot(g3_ref[...], d1 * d1, precision=_HI,
                            preferred_element_type=jnp.float32))
    dirs1 = d1 / (nrm1 + 1e-8)
    intens1 = _softplus(i1) + 0.2

    inv = pl.reciprocal(intens1 + 1e-8, approx=True)
    dir3 = jnp.dot(wnd_ref[...], dirs1,
                   preferred_element_type=jnp.float32)
    dir24 = jnp.dot(r24_ref[...], dir3, precision=_HI,
                    preferred_element_type=jnp.float32)

    np_cols, sp_cols, dmats = [], [], []
    for b in range(B):
        imgn = img[b] * inv[:, b:b + 1].reshape(C, 1, 1)
        dcol = dir24[:, b:b + 1]
        dmat = to_mat(dirs1[:, b:b + 1])
        dmats.append(dmat)
        dm0 = dmat[:, 0:1].reshape(F, 1, 1)
        dm1 = dmat[:, 1:2].reshape(F, 1, 1)
        dm2 = dmat[:, 2:3].reshape(F, 1, 1)
        np_acc = jnp.zeros((3, 1, 1), jnp.float32)
        sp_acc = jnp.zeros((F, 1, 1), jnp.float32)
        for k in range(H // 8):
            sl = slice(k * 8, (k + 1) * 8)
            chunk = imgn[:, sl, :].reshape(C * 8, W)
            raw = (jnp.dot(kw_ref[...], chunk,
                           preferred_element_type=jnp.float32)
                   + dcol)
            t3 = jnp.tanh(raw).reshape(3, 8, W)
            ssum = jnp.sum(t3 * t3, axis=0)
            scale = (jax.lax.rsqrt(ssum + 1e-8)
                     * mask[b * H + k * 8:b * H + (k + 1) * 8, :])
            normal_k = t3 * scale
            normal_ref[b, :, sl, :] = normal_k
            np_acc = np_acc + jnp.sum(normal_k, axis=(1, 2), keepdims=True)

            shad_k = (dm0 * normal_k[0:1] + dm1 * normal_k[1:2]
                      + dm2 * normal_k[2:3])
            shad_k = jnp.clip(shad_k, 0.0, 1.0)
            shading_ref[b, :, sl, :] = shad_k
            sp_acc = sp_acc + jnp.sum(shad_k, axis=(1, 2), keepdims=True)
        np_cols.append(np_acc.reshape(3, 1) * inv_hw)
        sp_cols.append(sp_acc.reshape(F, 1) * inv_hw)

    npool = jnp.concatenate(np_cols, axis=1)
    sp = jnp.concatenate(sp_cols, axis=1)

    h2 = (jnp.dot(w2_ref[...], pool, preferred_element_type=jnp.float32)
          + b2m_ref[...] * mpool
          + jnp.dot(w2n_ref[...], npool, preferred_element_type=jnp.float32)
          + jnp.dot(w2s_ref[...], sp, precision=_HI,
                    preferred_element_type=jnp.float32))
    d2, i2 = h2[0:C], h2[C:2 * C]
    nrm2 = jnp.sqrt(jnp.dot(g3_ref[...], d2 * d2, precision=_HI,
                            preferred_element_type=jnp.float32))
    dirs2 = d2 / (nrm2 + 1e-8)
    intens2 = _softplus(i2) + 0.2

    for b in range(B):
        small_ref[b] = jnp.concatenate(
            [dmats[b], to_mat(intens1[:, b:b + 1]),
             to_mat(dirs2[:, b:b + 1]), to_mat(intens2[:, b:b + 1])], axis=1)


def kernel(img, mask, l1_wd, l1_wi, n_w, l2_wd, l2_wi):
    N, c3f, H, W = img.shape
    F = c3f // 3
    C = 3 * F
    f32 = jnp.float32
    img = img.astype(f32)
    mask = mask.astype(f32)

    eyeF = jnp.eye(F, dtype=f32)
    eye3 = jnp.eye(3, dtype=f32)
    s24 = jnp.kron(jnp.eye(C, dtype=f32), jnp.ones((1, H), f32))
    w1 = jnp.concatenate([jnp.kron(eyeF, l1_wd[:, 0:3]),
                          jnp.kron(eyeF, l1_wi[:, 0:3])], axis=0)
    b1 = jnp.concatenate([jnp.tile(l1_wd[:, 3:4], (F, 1)),
                          jnp.tile(l1_wi[:, 3:4], (F, 1))], axis=0)
    g3 = jnp.kron(eyeF, jnp.ones((3, 3), f32))
    eye24 = jnp.eye(C, dtype=f32)

    n_wr = n_w.reshape(3, F, 6)
    wn3 = n_wr[:, :, 0:3].reshape(3, C)
    wnd3 = n_wr[:, :, 3:6].reshape(3, C)
    kw = jnp.kron(wn3, jnp.eye(8, dtype=f32))
    r24 = jnp.kron(eye3, jnp.ones((8, 1), f32))

    pmat = jnp.kron(eyeF, jnp.ones((1, 3), f32))
    qmat = jnp.kron(jnp.ones((F, 1), f32), eye3)

    w2 = jnp.concatenate([jnp.kron(eyeF, l2_wd[:, 0:3]),
                          jnp.kron(eyeF, l2_wi[:, 0:3])], axis=0)
    b2m = jnp.concatenate([jnp.tile(l2_wd[:, 3:4], (F, 1)),
                           jnp.tile(l2_wi[:, 3:4], (F, 1))], axis=0)
    w2n = jnp.concatenate([jnp.tile(l2_wd[:, 4:7], (F, 1)),
                           jnp.tile(l2_wi[:, 4:7], (F, 1))], axis=0)
    w2s = jnp.concatenate([jnp.kron(eyeF, l2_wd[:, 7:8]),
                           jnp.kron(eyeF, l2_wi[:, 7:8])], axis=0)

    def cspec(shape):
        return pl.BlockSpec(shape, lambda n: (0,) * len(shape))

    B = 4 if N % 4 == 0 else (2 if N % 2 == 0 else 1)

    normal, shading, small = pl.pallas_call(
        _gcnet_kernel,
        grid=(N // B,),
        in_specs=[
            pl.BlockSpec((B, C, H, W), lambda n: (n, 0, 0, 0)),
            pl.BlockSpec((B, 1, H, W), lambda n: (n, 0, 0, 0)),
            cspec((C, C * H)),
            cspec((2 * C, C)), cspec((2 * C, 1)),
            cspec((C, C)), cspec((C, C)),
            cspec((24, 8 * C)), cspec((3, C)), cspec((C, 3)),
            cspec((F, C)), cspec((C, 3)),
            cspec((2 * C, C)), cspec((2 * C, 1)),
            cspec((2 * C, 3)), cspec((2 * C, F)),
        ],
        out_specs=[
            pl.BlockSpec((B, 3, H, W), lambda n: (n, 0, 0, 0)),
            pl.BlockSpec((B, F, H, W), lambda n: (n, 0, 0, 0)),
            pl.BlockSpec((B, F, 12), lambda n: (n, 0, 0)),
        ],
        out_shape=[
            jax.ShapeDtypeStruct((N, 3, H, W), f32),
            jax.ShapeDtypeStruct((N, F, H, W), f32),
            jax.ShapeDtypeStruct((N, F, 12), f32),
        ],
        compiler_params=pltpu.CompilerParams(
            dimension_semantics=("parallel",)),
    )(img, mask, s24, w1, b1, g3, eye24, kw, wnd3, r24, pmat, qmat,
      w2, b2m, w2n, w2s)

    return {
        'prev_dirs': small[:, :, 0:3],
        'prev_intens': small[:, :, 3:6],
        'prev_normal': normal,
        'prev_shading': shading,
        'dirs': small[:, :, 6:9],
        'intens': small[:, :, 9:12],
    }


# speedup vs baseline: 4.4145x; 1.0700x over previous
import jax
import jax.numpy as jnp
from jax.experimental import pallas as pl
from jax.experimental.pallas import tpu as pltpu

_HI = jax.lax.Precision.HIGHEST


def _softplus(x):
    return jnp.maximum(x, 0.0) + jnp.log(1.0 + jnp.exp(-jnp.abs(x)))


def _gcnet_kernel(
        img_ref,
        mask_ref,
        s24_ref,
        w1_ref,
        b1_ref,
        g3_ref,
        eye_ref,
        kw_ref,
        wnd_ref,
        r24_ref,
        p_ref,
        q_ref,
        w2_ref,
        b2m_ref,
        w2n_ref,
        w2s_ref,
        normal_ref,
        shading_ref,
        small_ref):
    B, C, H, W = img_ref.shape
    F = C // 3
    inv_hw = 1.0 / (H * W)
    img = img_ref[...]
    mask = mask_ref[...].reshape(B * H, W)

    def to_mat(v):
        return jnp.dot(jnp.dot(p_ref[...], eye_ref[...] * v, precision=_HI,
                               preferred_element_type=jnp.float32),
                       q_ref[...], precision=_HI,
                       preferred_element_type=jnp.float32)

    pools, mpools = [], []
    for b in range(B):
        pw = jnp.dot(s24_ref[...], img[b].reshape(C * H, W),
                     preferred_element_type=jnp.float32)
        pools.append(jnp.sum(pw, axis=-1, keepdims=True) * inv_hw)
        mpools.append(jnp.mean(mask[b * H:(b + 1) * H, :], keepdims=True))
    pool = jnp.concatenate(pools, axis=1)
    mpool = jnp.concatenate(mpools, axis=1)

    h1 = (jnp.dot(w1_ref[...], pool, preferred_element_type=jnp.float32)
          + b1_ref[...] * mpool)
    d1, i1 = h1[0:C], h1[C:2 * C]
    nrm1 = jnp.sqrt(jnp.dot(g3_ref[...], d1 * d1, precision=_HI,
                            preferred_element_type=jnp.float32))
    dirs1 = d1 / (nrm1 + 1e-8)
    intens1 = _softplus(i1) + 0.2

    inv = pl.reciprocal(intens1 + 1e-8, approx=True)
    dir3 = jnp.dot(wnd_ref[...], dirs1,
                   preferred_element_type=jnp.float32)
    dir24 = jnp.dot(r24_ref[...], dir3, precision=_HI,
                    preferred_element_type=jnp.float32)

    np_cols, sp_cols, dmats = [], [], []
    for b in range(B):
        imgn = img[b] * inv[:, b:b + 1].reshape(C, 1, 1)
        dcol = dir24[:, b:b + 1]
        dmat = to_mat(dirs1[:, b:b + 1])
        dmats.append(dmat)
        dm0 = dmat[:, 0:1].reshape(F, 1, 1)
        dm1 = dmat[:, 1:2].reshape(F, 1, 1)
        dm2 = dmat[:, 2:3].reshape(F, 1, 1)
        np_acc = jnp.zeros((3, 1, 1), jnp.float32)
        sp_acc = jnp.zeros((F, 1, 1), jnp.float32)
        for k in range(H // 8):
            sl = slice(k * 8, (k + 1) * 8)
            chunk = imgn[:, sl, :].reshape(C * 8, W)
            raw = (jnp.dot(kw_ref[...], chunk,
                           preferred_element_type=jnp.float32)
                   + dcol)
            t3 = jnp.tanh(raw).reshape(3, 8, W)
            ssum = jnp.sum(t3 * t3, axis=0)
            scale = (jax.lax.rsqrt(ssum + 1e-8)
                     * mask[b * H + k * 8:b * H + (k + 1) * 8, :])
            normal_k = t3 * scale
            normal_ref[b, :, sl, :] = normal_k
            np_acc = np_acc + jnp.sum(normal_k, axis=(1, 2), keepdims=True)

            shad_k = (dm0 * normal_k[0:1] + dm1 * normal_k[1:2]
                      + dm2 * normal_k[2:3])
            shad_k = jnp.clip(shad_k, 0.0, 1.0)
            shading_ref[b, :, sl, :] = shad_k
            sp_acc = sp_acc + jnp.sum(shad_k, axis=(1, 2), keepdims=True)
        np_cols.append(np_acc.reshape(3, 1) * inv_hw)
        sp_cols.append(sp_acc.reshape(F, 1) * inv_hw)

    npool = jnp.concatenate(np_cols, axis=1)
    sp = jnp.concatenate(sp_cols, axis=1)

    h2 = (jnp.dot(w2_ref[...], pool, preferred_element_type=jnp.float32)
          + b2m_ref[...] * mpool
          + jnp.dot(w2n_ref[...], npool, preferred_element_type=jnp.float32)
          + jnp.dot(w2s_ref[...], sp, precision=_HI,
                    preferred_element_type=jnp.float32))
    d2, i2 = h2[0:C], h2[C:2 * C]
    nrm2 = jnp.sqrt(jnp.dot(g3_ref[...], d2 * d2, precision=_HI,
                            preferred_element_type=jnp.float32))
    dirs2 = d2 / (nrm2 + 1e-8)
    intens2 = _softplus(i2) + 0.2

    for b in range(B):
        small_ref[b] = jnp.concatenate(
            [dmats[b], to_mat(intens1[:, b:b + 1]),
             to_mat(dirs2[:, b:b + 1]), to_mat(intens2[:, b:b + 1])], axis=1)


def kernel(img, mask, l1_wd, l1_wi, n_w, l2_wd, l2_wi):
    N, c3f, H, W = img.shape
    F = c3f // 3
    C = 3 * F
    f32 = jnp.float32
    img = img.astype(f32)
    mask = mask.astype(f32)

    eyeF = jnp.eye(F, dtype=f32)
    eye3 = jnp.eye(3, dtype=f32)
    s24 = jnp.kron(jnp.eye(C, dtype=f32), jnp.ones((1, H), f32))
    w1 = jnp.concatenate([jnp.kron(eyeF, l1_wd[:, 0:3]),
                          jnp.kron(eyeF, l1_wi[:, 0:3])], axis=0)
    b1 = jnp.concatenate([jnp.tile(l1_wd[:, 3:4], (F, 1)),
                          jnp.tile(l1_wi[:, 3:4], (F, 1))], axis=0)
    g3 = jnp.kron(eyeF, jnp.ones((3, 3), f32))
    eye24 = jnp.eye(C, dtype=f32)

    n_wr = n_w.reshape(3, F, 6)
    wn3 = n_wr[:, :, 0:3].reshape(3, C)
    wnd3 = n_wr[:, :, 3:6].reshape(3, C)
    kw = jnp.kron(wn3, jnp.eye(8, dtype=f32))
    r24 = jnp.kron(eye3, jnp.ones((8, 1), f32))

    pmat = jnp.kron(eyeF, jnp.ones((1, 3), f32))
    qmat = jnp.kron(jnp.ones((F, 1), f32), eye3)

    w2 = jnp.concatenate([jnp.kron(eyeF, l2_wd[:, 0:3]),
                          jnp.kron(eyeF, l2_wi[:, 0:3])], axis=0)
    b2m = jnp.concatenate([jnp.tile(l2_wd[:, 3:4], (F, 1)),
                           jnp.tile(l2_wi[:, 3:4], (F, 1))], axis=0)
    w2n = jnp.concatenate([jnp.tile(l2_wd[:, 4:7], (F, 1)),
                           jnp.tile(l2_wi[:, 4:7], (F, 1))], axis=0)
    w2s = jnp.concatenate([jnp.kron(eyeF, l2_wd[:, 7:8]),
                           jnp.kron(eyeF, l2_wi[:, 7:8])], axis=0)

    def cspec(shape):
        return pl.BlockSpec(shape, lambda n: (0,) * len(shape))

    B = 8 if N % 8 == 0 else (2 if N % 2 == 0 else 1)

    normal, shading, small = pl.pallas_call(
        _gcnet_kernel,
        grid=(N // B,),
        in_specs=[
            pl.BlockSpec((B, C, H, W), lambda n: (n, 0, 0, 0)),
            pl.BlockSpec((B, 1, H, W), lambda n: (n, 0, 0, 0)),
            cspec((C, C * H)),
            cspec((2 * C, C)), cspec((2 * C, 1)),
            cspec((C, C)), cspec((C, C)),
            cspec((24, 8 * C)), cspec((3, C)), cspec((C, 3)),
            cspec((F, C)), cspec((C, 3)),
            cspec((2 * C, C)), cspec((2 * C, 1)),
            cspec((2 * C, 3)), cspec((2 * C, F)),
        ],
        out_specs=[
            pl.BlockSpec((B, 3, H, W), lambda n: (n, 0, 0, 0)),
            pl.BlockSpec((B, F, H, W), lambda n: (n, 0, 0, 0)),
            pl.BlockSpec((B, F, 12), lambda n: (n, 0, 0)),
        ],
        out_shape=[
            jax.ShapeDtypeStruct((N, 3, H, W), f32),
            jax.ShapeDtypeStruct((N, F, H, W), f32),
            jax.ShapeDtypeStruct((N, F, 12), f32),
        ],
        compiler_params=pltpu.CompilerParams(
            dimension_semantics=("parallel",)),
    )(img, mask, s24, w1, b1, g3, eye24, kw, wnd3, r24, pmat, qmat,
      w2, b2m, w2n, w2s)

    return {
        'prev_dirs': small[:, :, 0:3],
        'prev_intens': small[:, :, 3:6],
        'prev_normal': normal,
        'prev_shading': shading,
        'dirs': small[:, :, 6:9],
        'intens': small[:, :, 9:12],
    }


# speedup vs baseline: 5.9838x; 1.3555x over previous
import functools

import jax
import jax.numpy as jnp
from jax.experimental import pallas as pl
from jax.experimental.pallas import tpu as pltpu

_HI = jax.lax.Precision.HIGHEST


def _softplus(x):
    return jnp.maximum(x, 0.0) + jnp.log(1.0 + jnp.exp(-jnp.abs(x)))


def _hidot(a, b):
    return jnp.dot(a, b, precision=_HI, preferred_element_type=jnp.float32)


def _gcnet_kernel(
        img_ref,
        mask_ref,
        l1_ref,
        l1i_ref,
        nw_ref,
        l2_ref,
        l2i_ref,
        g3_ref,
        eye_ref,
        qmat_ref,
        qt_ref,
        p_ref,
        p8t_ref,
        s8_ref,
        k8m_ref,
        selimg_ref,
        seldir_ref,
        repf_ref,
        normal_ref,
        shading_ref,
        dirs1_ref,
        intens1_ref,
        dirs2_ref,
        intens2_ref):
    B, C, H, W = img_ref.shape
    F = C // 3
    inv_hw = 1.0 / (H * W)
    img = img_ref[...]
    mask = mask_ref[...].reshape(B * H, W)
    qmat, g3, eye = qmat_ref[...], g3_ref[...], eye_ref[...]

    def to_mat(v):
        return _hidot(_hidot(p_ref[...], eye * v), qmat)

    l1d, l1i = l1_ref[...], l1i_ref[...]
    l2d, l2i = l2_ref[...], l2i_ref[...]

    def blockdiag(a3):
        return _hidot(_hidot(qmat, a3), qt_ref[...]) * g3

    w1 = jnp.concatenate([blockdiag(l1d[:, 0:3]),
                          blockdiag(l1i[:, 0:3])], axis=0)
    b1 = jnp.concatenate([_hidot(qmat, l1d[:, 3:4]),
                          _hidot(qmat, l1i[:, 3:4])], axis=0)
    w2 = jnp.concatenate([blockdiag(l2d[:, 0:3]),
                          blockdiag(l2i[:, 0:3])], axis=0)
    b2m = jnp.concatenate([_hidot(qmat, l2d[:, 3:4]),
                           _hidot(qmat, l2i[:, 3:4])], axis=0)
    w2n = jnp.concatenate([_hidot(qmat, l2d[:, 4:7]),
                           _hidot(qmat, l2i[:, 4:7])], axis=0)
    w2s = jnp.concatenate([_hidot(qmat, l2d[:, 7:8]) * repf_ref[...],
                           _hidot(qmat, l2i[:, 7:8]) * repf_ref[...]],
                          axis=0)
    wn3 = _hidot(nw_ref[...], selimg_ref[...])
    wnd3 = _hidot(nw_ref[...], seldir_ref[...])
    kw = _hidot(_hidot(p8t_ref[...], wn3), s8_ref[...]) * k8m_ref[...]

    pools, mpools = [], []
    for b in range(B):
        pw = jnp.sum(img[b], axis=1)
        pools.append(jnp.sum(pw, axis=-1, keepdims=True) * inv_hw)
        mpools.append(jnp.mean(mask[b * H:(b + 1) * H, :], keepdims=True))
    pool = jnp.concatenate(pools, axis=1)
    mpool = jnp.concatenate(mpools, axis=1)

    h1 = (jnp.dot(w1, pool, preferred_element_type=jnp.float32)
          + b1 * mpool)
    d1, i1 = h1[0:C], h1[C:2 * C]
    nrm1 = jnp.sqrt(_hidot(g3, d1 * d1))
    dirs1 = d1 / (nrm1 + 1e-8)
    intens1 = _softplus(i1) + 0.2

    inv = pl.reciprocal(intens1 + 1e-8, approx=True)
    dir3 = jnp.dot(wnd3, dirs1,
                   preferred_element_type=jnp.float32)
    dir24 = _hidot(p8t_ref[...], dir3)

    np_cols, sp_cols = [], []
    for b in range(B):
        imgn = img[b] * inv[:, b:b + 1].reshape(C, 1, 1)
        dcol = dir24[:, b:b + 1]
        dmat = to_mat(dirs1[:, b:b + 1])
        dirs1_ref[b] = dmat
        intens1_ref[b] = to_mat(intens1[:, b:b + 1])
        dm0 = dmat[:, 0:1].reshape(F, 1, 1)
        dm1 = dmat[:, 1:2].reshape(F, 1, 1)
        dm2 = dmat[:, 2:3].reshape(F, 1, 1)
        np_acc = jnp.zeros((3, 1, 1), jnp.float32)
        sp_acc = jnp.zeros((F, 1, 1), jnp.float32)
        for k in range(H // 8):
            sl = slice(k * 8, (k + 1) * 8)
            chunk = imgn[:, sl, :].reshape(C * 8, W)
            raw = (jnp.dot(kw, chunk,
                           preferred_element_type=jnp.float32)
                   + dcol)
            t3 = jnp.tanh(raw).reshape(3, 8, W)
            ssum = jnp.sum(t3 * t3, axis=0)
            scale = (jax.lax.rsqrt(ssum + 1e-8)
                     * mask[b * H + k * 8:b * H + (k + 1) * 8, :])
            normal_k = t3 * scale
            normal_ref[b, :, sl, :] = normal_k
            np_acc = np_acc + jnp.sum(normal_k, axis=(1, 2), keepdims=True)

            shad_k = (dm0 * normal_k[0:1] + dm1 * normal_k[1:2]
                      + dm2 * normal_k[2:3])
            shad_k = jnp.clip(shad_k, 0.0, 1.0)
            shading_ref[b, :, sl, :] = shad_k
            sp_acc = sp_acc + jnp.sum(shad_k, axis=(1, 2), keepdims=True)
        np_cols.append(np_acc.reshape(3, 1) * inv_hw)
        sp_cols.append(sp_acc.reshape(F, 1) * inv_hw)

    npool = jnp.concatenate(np_cols, axis=1)
    sp = jnp.concatenate(sp_cols, axis=1)

    h2 = (jnp.dot(w2, pool, preferred_element_type=jnp.float32)
          + b2m * mpool
          + jnp.dot(w2n, npool, preferred_element_type=jnp.float32)
          + _hidot(w2s, sp))
    d2, i2 = h2[0:C], h2[C:2 * C]
    nrm2 = jnp.sqrt(_hidot(g3, d2 * d2))
    dirs2 = d2 / (nrm2 + 1e-8)
    intens2 = _softplus(i2) + 0.2
    for b in range(B):
        dirs2_ref[b] = to_mat(dirs2[:, b:b + 1])
        intens2_ref[b] = to_mat(intens2[:, b:b + 1])


def kernel(img, mask, l1_wd, l1_wi, n_w, l2_wd, l2_wi):
    N, c3f, H, W = img.shape
    F = c3f // 3
    C = 3 * F
    f32 = jnp.float32
    img = img.astype(f32)
    mask = mask.astype(f32)

    import numpy as np
    nf32 = np.float32
    eyeF = np.eye(F, dtype=nf32)
    eye3 = np.eye(3, dtype=nf32)
    g3 = jnp.asarray(np.kron(eyeF, np.ones((3, 3), nf32)))
    eye24 = jnp.asarray(np.eye(C, dtype=nf32))
    qmat_n = np.kron(np.ones((F, 1), nf32), eye3)
    pmat_n = np.kron(eyeF, np.ones((1, 3), nf32))
    qmat, qt = jnp.asarray(qmat_n), jnp.asarray(qmat_n.T)
    pmat = jnp.asarray(pmat_n)
    p8t = jnp.asarray(np.kron(eye3, np.ones((8, 1), nf32)))
    s8 = jnp.asarray(np.kron(np.eye(C, dtype=nf32),
                             np.ones((1, 8), nf32)))
    k8m = jnp.asarray(np.kron(np.ones((3, C), nf32),
                              np.eye(8, dtype=nf32)))
    selimg_n = np.zeros((6 * F, C), nf32)
    seldir_n = np.zeros((6 * F, C), nf32)
    for f in range(F):
        for j in range(3):
            selimg_n[6 * f + j, 3 * f + j] = 1.0
            seldir_n[6 * f + 3 + j, 3 * f + j] = 1.0
    selimg, seldir = jnp.asarray(selimg_n), jnp.asarray(seldir_n)
    repf = jnp.asarray(np.kron(eyeF, np.ones((3, 1), nf32)))

    def cspec(shape):
        return pl.BlockSpec(shape, lambda n: (0,) * len(shape))

    B = 8 if N % 8 == 0 else (2 if N % 2 == 0 else 1)

    outs = pl.pallas_call(
        _gcnet_kernel,
        grid=(N // B,),
        in_specs=[
            pl.BlockSpec((B, C, H, W), lambda n: (n, 0, 0, 0)),
            pl.BlockSpec((B, 1, H, W), lambda n: (n, 0, 0, 0)),
            cspec((3, 4)), cspec((3, 4)),
            cspec((3, 6 * F)),
            cspec((3, 8)), cspec((3, 8)),
            cspec((C, C)), cspec((C, C)),
            cspec((C, 3)), cspec((3, C)),
            cspec((F, C)),
            cspec((C, 3)),
            cspec((C, 8 * C)), cspec((C, 8 * C)),
            cspec((6 * F, C)), cspec((6 * F, C)),
            cspec((C, F)),
        ],
        out_specs=[
            pl.BlockSpec((B, 3, H, W), lambda n: (n, 0, 0, 0)),
            pl.BlockSpec((B, F, H, W), lambda n: (n, 0, 0, 0)),
            pl.BlockSpec((B, F, 3), lambda n: (n, 0, 0)),
            pl.BlockSpec((B, F, 3), lambda n: (n, 0, 0)),
            pl.BlockSpec((B, F, 3), lambda n: (n, 0, 0)),
            pl.BlockSpec((B, F, 3), lambda n: (n, 0, 0)),
        ],
        out_shape=[
            jax.ShapeDtypeStruct((N, 3, H, W), f32),
            jax.ShapeDtypeStruct((N, F, H, W), f32),
            jax.ShapeDtypeStruct((N, F, 3), f32),
            jax.ShapeDtypeStruct((N, F, 3), f32),
            jax.ShapeDtypeStruct((N, F, 3), f32),
            jax.ShapeDtypeStruct((N, F, 3), f32),
        ],
        compiler_params=pltpu.CompilerParams(
            dimension_semantics=("parallel",)),
    )(img, mask, l1_wd.astype(f32), l1_wi.astype(f32), n_w.astype(f32),
      l2_wd.astype(f32), l2_wi.astype(f32),
      g3, eye24, qmat, qt, pmat, p8t, s8, k8m, selimg, seldir, repf)

    normal, shading, dirs1, intens1, dirs2, intens2 = outs
    return {
        'prev_dirs': dirs1,
        'prev_intens': intens1,
        'prev_normal': normal,
        'prev_shading': shading,
        'dirs': dirs2,
        'intens': intens2,
    }
